# baseline (speedup 1.0000x reference)
import functools

import jax
import jax.numpy as jnp
from jax import lax
from jax.experimental import pallas as pl
from jax.experimental.pallas import tpu as pltpu
from jax.experimental.pallas import tpu_sc as plsc

N_NODES = 10000
N_EDGES = 320000
NC = 2
NS = 16
L = 16
EDGES_PER_WORKER = N_EDGES // (NC * NS)
B = 40
NBLK = EDGES_PER_WORKER // B
SBK = 10
NSB = NBLK // SBK
NZCHUNK = N_NODES // B
ACC_W = 144


def _build_wbig(W0, W1, Ws0, Ws1):
    c1 = 1.0 / (1.5 * jnp.sqrt(96.0))
    c2 = 1.0 / jnp.sqrt(32.0)
    A0, A1, A2 = W0[0:32], W0[32:64], W0[64:96]
    B0, B1, B2 = W1[0:32], W1[32:64], W1[64:96]
    z = jnp.zeros((32, 32), jnp.float32)
    CB = c1 * B0 + c2 * Ws1
    E1 = c1 * B1
    D2 = (c1 / jnp.sqrt(3.0)) * A2
    r_x0 = jnp.concatenate(
        [c1 * A0 + c2 * Ws0, z, z, z, c1 * A1, z, z, z, z, z, z, c1 * B2],
        axis=1)
    rows = [r_x0]
    for j in range(3):
        blocks = [z] * 12
        blocks[1 + j] = CB
        blocks[5 + j] = E1
        blocks[8 + j] = D2
        rows.append(jnp.concatenate(blocks, axis=1))
    return jnp.concatenate(rows, axis=0)


def _matmul_body(x_ref, w_ref, o_ref):
    o_ref[...] = jnp.dot(x_ref[...], w_ref[...],
                         preferred_element_type=jnp.float32)


def _precompute_p(x, wbig):
    blk = 400
    grid = (N_NODES // blk,)
    return pl.pallas_call(
        _matmul_body,
        grid=grid,
        in_specs=[
            pl.BlockSpec((blk, 128), lambda i: (i, 0)),
            pl.BlockSpec((128, 384), lambda i: (0, 0)),
        ],
        out_specs=pl.BlockSpec((blk, 384), lambda i: (i, 0)),
        out_shape=jax.ShapeDtypeStruct((N_NODES, 384), jnp.float32),
    )(x, wbig)


def _epilogue_body(p_ref, o_ref):
    s = p_ref[0] + p_ref[1]
    cnt = jnp.maximum(s[:, 128:129], 1.0)
    o_ref[...] = s[:, 0:128] / cnt


def _epilogue(parts):
    blk = 400
    grid = (N_NODES // blk,)
    return pl.pallas_call(
        _epilogue_body,
        grid=grid,
        in_specs=[pl.BlockSpec((2, blk, ACC_W), lambda i: (0, i, 0))],
        out_specs=pl.BlockSpec((blk, 128), lambda i: (i, 0)),
        out_shape=jax.ShapeDtypeStruct((N_NODES, 128), jnp.float32),
    )(parts)


def _splat(shc, j, col):
    ji = jnp.full((L,), j, jnp.int32)
    ci = jnp.full((L,), col, jnp.int32)
    return plsc.load_gather(shc, [ji, ci])


def _sc_body(p_hbm, send_hbm, recv_hbm, sh_hbm, out_hbm,
             sidxb0, sidxb1, ridxb0, ridxb1, shc, pbuf0, pbuf1, ybuf, acc,
             gsem0, gsem1, isem0, isem1, ssem):
    c = lax.axis_index("c")
    s = lax.axis_index("s")
    wid = c * NS + s
    blkbase = wid * NBLK

    zvec = jnp.zeros((L,), jnp.float32)
    for r in range(B):
        for k in range(ACC_W // L):
            ybuf[r, pl.ds(k * L, L)] = zvec

    def zloop(k, carry):
        cidx = k * NS + s

        @pl.when(cidx < NZCHUNK)
        def _():
            pltpu.sync_copy(ybuf, acc.at[pl.ds(cidx * B, B)])
        return carry
    lax.fori_loop(0, (NZCHUNK + NS - 1) // NS, zloop, 0)
    plsc.subcore_barrier()

    lane = lax.iota(jnp.int32, L)
    ones_chunk = jnp.where(lane == 0, 1.0, 0.0).astype(jnp.float32)
    pbufs = (pbuf0, pbuf1)
    gsems = (gsem0, gsem1)
    sidxbs = (sidxb0, sidxb1)
    ridxbs = (ridxb0, ridxb1)
    isems = (isem0, isem1)

    def compute_block(pbuf, j):
        for q in range(B // 4):
            for r4 in range(4):
                e = 4 * q + r4
                col = 16 * q + 4 * r4
                sh_l0 = _splat(shc, j, col)
                s1_a = _splat(shc, j, col + 1)
                s1_b = _splat(shc, j, col + 2)
                s1_c = _splat(shc, j, col + 3)
                t0 = pbuf[e, pl.ds(352, L)]
                t1 = pbuf[e, pl.ds(368, L)]
                for k in range(2):
                    off = k * L
                    y = (pbuf[e, pl.ds(off, L)]
                         + sh_l0 * pbuf[e, pl.ds(128 + off, L)]
                         + s1_a * pbuf[e, pl.ds(256 + off, L)]
                         + s1_b * pbuf[e, pl.ds(288 + off, L)]
                         + s1_c * pbuf[e, pl.ds(320 + off, L)])
                    ybuf[e, pl.ds(off, L)] = y
                for jj, sj in ((0, s1_a), (1, s1_b), (2, s1_c)):
                    for m, tm in ((0, t0), (1, t1)):
                        off = 32 + 32 * jj + m * L
                        y = (pbuf[e, pl.ds(off, L)]
                             + sh_l0 * pbuf[e, pl.ds(128 + off, L)]
                             + sj * tm)
                        ybuf[e, pl.ds(off, L)] = y
                ybuf[e, pl.ds(128, L)] = ones_chunk

    def fetch_idx(sb, slot):
        rowbase = blkbase + sb * SBK
        pltpu.async_copy(send_hbm.at[pl.ds(rowbase, SBK)],
                         sidxbs[slot], isems[slot])
        pltpu.async_copy(recv_hbm.at[pl.ds(rowbase, SBK)],
                         ridxbs[slot], isems[slot])

    def wait_idx(sb, slot):
        rowbase = blkbase + sb * SBK
        pltpu.make_async_copy(send_hbm.at[pl.ds(rowbase, SBK)],
                              sidxbs[slot], isems[slot]).wait()
        pltpu.make_async_copy(recv_hbm.at[pl.ds(rowbase, SBK)],
                              ridxbs[slot], isems[slot]).wait()

    fetch_idx(0, 0)

    def sb_exec(sb, slot):
        sidxb = sidxbs[slot]
        ridxb = ridxbs[slot]
        wait_idx(sb, slot)

        @pl.when(sb + 1 < NSB)
        def _():
            fetch_idx(sb + 1, 1 - slot)
        rowbase = blkbase + sb * SBK
        pltpu.sync_copy(sh_hbm.at[pl.ds(rowbase, SBK)], shc)
        pltpu.async_copy(p_hbm.at[sidxb.at[0]], pbuf0, gsem0)
        pltpu.async_copy(p_hbm.at[sidxb.at[1]], pbuf1, gsem1)

        def jj_body(jj, carry2):
            for phase in range(2):
                j = jj * 2 + phase
                pltpu.make_async_copy(p_hbm.at[sidxb.at[j]],
                                      pbufs[phase], gsems[phase]).wait()

                @pl.when(sb * SBK + j > 0)
                def _():
                    pltpu.make_async_copy(ybuf, acc.at[ridxb.at[j]],
                                          ssem).wait()
                compute_block(pbufs[phase], j)
                pltpu.async_copy(ybuf, acc.at[ridxb.at[j]], ssem,
                                 add=True)

                @pl.when(j + 2 < SBK)
                def _():
                    pltpu.async_copy(p_hbm.at[sidxb.at[j + 2]],
                                     pbufs[phase], gsems[phase])
            return carry2
        lax.fori_loop(0, SBK // 2, jj_body, 0)

    def sb2_body(sb2, carry):
        for slot in range(2):
            sb_exec(sb2 * 2 + slot, slot)
        return carry
    lax.fori_loop(0, NSB // 2, sb2_body, 0)
    sb_exec(jnp.int32(NSB - 1), 0)

    pltpu.make_async_copy(ybuf, acc.at[sidxbs[0].at[SBK - 1]], ssem).wait()

    plsc.subcore_barrier()
    rpt = N_NODES // NS
    pltpu.sync_copy(acc.at[pl.ds(s * rpt, rpt)],
                    out_hbm.at[c, pl.ds(s * rpt, rpt)])


def _sc_aggregate(p, send2d, recv2d, sh2d):
    mesh = plsc.VectorSubcoreMesh(core_axis_name="c", subcore_axis_name="s")
    kern = pl.kernel(
        _sc_body,
        out_type=jax.ShapeDtypeStruct((NC, N_NODES, ACC_W), jnp.float32),
        mesh=mesh,
        compiler_params=pltpu.CompilerParams(use_tc_tiling_on_sc=False,
                                             needs_layout_passes=False),
        scratch_types=[
            pltpu.VMEM((SBK, B), jnp.int32),
            pltpu.VMEM((SBK, B), jnp.int32),
            pltpu.VMEM((SBK, B), jnp.int32),
            pltpu.VMEM((SBK, B), jnp.int32),
            pltpu.VMEM((SBK, 4 * B), jnp.float32),
            pltpu.VMEM((B, 384), jnp.float32),
            pltpu.VMEM((B, 384), jnp.float32),
            pltpu.VMEM((B, ACC_W), jnp.float32),
            pltpu.VMEM_SHARED((N_NODES, ACC_W), jnp.float32),
            pltpu.SemaphoreType.DMA,
            pltpu.SemaphoreType.DMA,
            pltpu.SemaphoreType.DMA,
            pltpu.SemaphoreType.DMA,
            pltpu.SemaphoreType.DMA,
        ],
    )
    return kern(p, send2d, recv2d, sh2d)


def kernel(node_features, relative_positions_sh, senders, receivers,
           W0, W1, Ws0, Ws1):
    senders = senders.astype(jnp.int32).reshape(N_EDGES // B, B)
    receivers = receivers.astype(jnp.int32).reshape(N_EDGES // B, B)
    sh2d = relative_positions_sh.astype(jnp.float32).reshape(
        N_EDGES // B, 4 * B)
    wbig = _build_wbig(W0, W1, Ws0, Ws1)
    p = _precompute_p(node_features, wbig)
    parts = _sc_aggregate(p, senders, receivers, sh2d)
    return _epilogue(parts)


# speedup vs baseline: 1.1832x; 1.1832x over previous
import functools

import jax
import jax.numpy as jnp
from jax import lax
from jax.experimental import pallas as pl
from jax.experimental.pallas import tpu as pltpu
from jax.experimental.pallas import tpu_sc as plsc

N_NODES = 10000
N_EDGES = 320000
NC = 2
NS = 16
L = 16
EDGES_PER_WORKER = N_EDGES // (NC * NS)
B = 40
NBLK = EDGES_PER_WORKER // B
SBK = 10
NSB = NBLK // SBK
NZCHUNK = N_NODES // B
ACC_W = 144


def _build_wbig(W0, W1, Ws0, Ws1):
    c1 = 1.0 / (1.5 * jnp.sqrt(96.0))
    c2 = 1.0 / jnp.sqrt(32.0)
    A0, A1, A2 = W0[0:32], W0[32:64], W0[64:96]
    B0, B1, B2 = W1[0:32], W1[32:64], W1[64:96]
    z = jnp.zeros((32, 32), jnp.float32)
    CB = c1 * B0 + c2 * Ws1
    E1 = c1 * B1
    D2 = (c1 / jnp.sqrt(3.0)) * A2
    r_x0 = jnp.concatenate(
        [c1 * A0 + c2 * Ws0, z, z, z, c1 * A1, z, z, z, z, z, z, c1 * B2],
        axis=1)
    rows = [r_x0]
    for j in range(3):
        blocks = [z] * 12
        blocks[1 + j] = CB
        blocks[5 + j] = E1
        blocks[8 + j] = D2
        rows.append(jnp.concatenate(blocks, axis=1))
    return jnp.concatenate(rows, axis=0)


def _matmul_body(x_ref, w_ref, o_ref):
    o_ref[...] = jnp.dot(x_ref[...], w_ref[...],
                         preferred_element_type=jnp.float32)


def _precompute_p(x, wbig):
    blk = 400
    grid = (N_NODES // blk,)
    return pl.pallas_call(
        _matmul_body,
        grid=grid,
        in_specs=[
            pl.BlockSpec((blk, 128), lambda i: (i, 0)),
            pl.BlockSpec((128, 384), lambda i: (0, 0)),
        ],
        out_specs=pl.BlockSpec((blk, 384), lambda i: (i, 0)),
        out_shape=jax.ShapeDtypeStruct((N_NODES, 384), jnp.float32),
    )(x, wbig)


def _epilogue_body(p_ref, o_ref):
    s = p_ref[0] + p_ref[1]
    cnt = jnp.maximum(s[:, 128:129], 1.0)
    o_ref[...] = s[:, 0:128] / cnt


def _epilogue(parts):
    blk = 400
    grid = (N_NODES // blk,)
    return pl.pallas_call(
        _epilogue_body,
        grid=grid,
        in_specs=[pl.BlockSpec((2, blk, ACC_W), lambda i: (0, i, 0))],
        out_specs=pl.BlockSpec((blk, 128), lambda i: (i, 0)),
        out_shape=jax.ShapeDtypeStruct((N_NODES, 128), jnp.float32),
    )(parts)


def _splat(shc, k, col):
    ki = jnp.full((L,), k, jnp.int32)
    ci = jnp.full((L,), col, jnp.int32)
    return plsc.load_gather(shc, [ki, ci])


def _sc_body(p_hbm, send_hbm, recv_hbm, sh_hbm, out_hbm,
             sidxb0, sidxb1, ridxb0, ridxb1, shc, pbuf0, pbuf1, ybuf, acc,
             gsem0, gsem1, isem0, isem1):
    c = lax.axis_index("c")
    s = lax.axis_index("s")
    wid = c * NS + s
    blkbase = wid * NBLK

    zvec = jnp.zeros((L,), jnp.float32)
    for r in range(B):
        for k in range(ACC_W // L):
            ybuf[r, pl.ds(k * L, L)] = zvec

    def zloop(k, carry):
        cidx = k * NS + s

        @pl.when(cidx < NZCHUNK)
        def _():
            pltpu.sync_copy(ybuf, acc.at[pl.ds(cidx * B, B)])
        return carry
    lax.fori_loop(0, (NZCHUNK + NS - 1) // NS, zloop, 0)

    lane = lax.iota(jnp.int32, L)
    ones_chunk = jnp.where(lane == 0, 1.0, 0.0).astype(jnp.float32)
    for r in range(B):
        ybuf[r, pl.ds(128, L)] = ones_chunk
    plsc.subcore_barrier()

    pbufs = (pbuf0, pbuf1)
    gsems = (gsem0, gsem1)
    sidxbs = (sidxb0, sidxb1)
    ridxbs = (ridxb0, ridxb1)
    isems = (isem0, isem1)

    def compute_block(pbuf, j):
        for e in range(B):
            col = j * B + e
            sh_l0 = _splat(shc, 0, col)
            s1_a = _splat(shc, 1, col)
            s1_b = _splat(shc, 2, col)
            s1_c = _splat(shc, 3, col)
            t0 = pbuf[e, pl.ds(352, L)]
            t1 = pbuf[e, pl.ds(368, L)]
            for k in range(2):
                off = k * L
                y = (pbuf[e, pl.ds(off, L)]
                     + sh_l0 * pbuf[e, pl.ds(128 + off, L)]
                     + s1_a * pbuf[e, pl.ds(256 + off, L)]
                     + s1_b * pbuf[e, pl.ds(288 + off, L)]
                     + s1_c * pbuf[e, pl.ds(320 + off, L)])
                ybuf[e, pl.ds(off, L)] = y
            for jj, sj in ((0, s1_a), (1, s1_b), (2, s1_c)):
                for m, tm in ((0, t0), (1, t1)):
                    off = 32 + 32 * jj + m * L
                    y = (pbuf[e, pl.ds(off, L)]
                         + sh_l0 * pbuf[e, pl.ds(128 + off, L)]
                         + sj * tm)
                    ybuf[e, pl.ds(off, L)] = y

    def fetch_idx(sb, slot):
        rowbase = blkbase + sb * SBK
        pltpu.async_copy(send_hbm.at[pl.ds(rowbase, SBK)],
                         sidxbs[slot], isems[slot])
        pltpu.async_copy(recv_hbm.at[pl.ds(rowbase, SBK)],
                         ridxbs[slot], isems[slot])

    def wait_idx(sb, slot):
        rowbase = blkbase + sb * SBK
        pltpu.make_async_copy(send_hbm.at[pl.ds(rowbase, SBK)],
                              sidxbs[slot], isems[slot]).wait()
        pltpu.make_async_copy(recv_hbm.at[pl.ds(rowbase, SBK)],
                              ridxbs[slot], isems[slot]).wait()

    fetch_idx(0, 0)

    def sb_exec(sb, slot):
        sidxb = sidxbs[slot]
        ridxb = ridxbs[slot]
        wait_idx(sb, slot)

        @pl.when(sb + 1 < NSB)
        def _():
            fetch_idx(sb + 1, 1 - slot)
        ebase = (blkbase + sb * SBK) * B
        pltpu.sync_copy(sh_hbm.at[:, pl.ds(ebase, SBK * B)], shc)
        pltpu.async_copy(p_hbm.at[sidxb.at[0]], pbuf0, gsem0)
        pltpu.async_copy(p_hbm.at[sidxb.at[1]], pbuf1, gsem1)

        def jj_body(jj, carry2):
            for phase in range(2):
                j = jj * 2 + phase
                pltpu.make_async_copy(p_hbm.at[sidxb.at[j]],
                                      pbufs[phase], gsems[phase]).wait()
                compute_block(pbufs[phase], j)
                pltpu.sync_copy(ybuf, acc.at[ridxb.at[j]], add=True)

                @pl.when(j + 2 < SBK)
                def _():
                    pltpu.async_copy(p_hbm.at[sidxb.at[j + 2]],
                                     pbufs[phase], gsems[phase])
            return carry2
        lax.fori_loop(0, SBK // 2, jj_body, 0)

    def sb2_body(sb2, carry):
        for slot in range(2):
            sb_exec(sb2 * 2 + slot, slot)
        return carry
    lax.fori_loop(0, NSB // 2, sb2_body, 0)
    sb_exec(jnp.int32(NSB - 1), 0)

    plsc.subcore_barrier()
    rpt = N_NODES // NS
    pltpu.sync_copy(acc.at[pl.ds(s * rpt, rpt)],
                    out_hbm.at[c, pl.ds(s * rpt, rpt)])


def _sc_aggregate(p, send2d, recv2d, sh2d):
    mesh = plsc.VectorSubcoreMesh(core_axis_name="c", subcore_axis_name="s")
    kern = pl.kernel(
        _sc_body,
        out_type=jax.ShapeDtypeStruct((NC, N_NODES, ACC_W), jnp.float32),
        mesh=mesh,
        compiler_params=pltpu.CompilerParams(use_tc_tiling_on_sc=False,
                                             needs_layout_passes=False),
        scratch_types=[
            pltpu.VMEM((SBK, B), jnp.int32),
            pltpu.VMEM((SBK, B), jnp.int32),
            pltpu.VMEM((SBK, B), jnp.int32),
            pltpu.VMEM((SBK, B), jnp.int32),
            pltpu.VMEM((4, SBK * B), jnp.float32),
            pltpu.VMEM((B, 384), jnp.float32),
            pltpu.VMEM((B, 384), jnp.float32),
            pltpu.VMEM((B, ACC_W), jnp.float32),
            pltpu.VMEM_SHARED((N_NODES, ACC_W), jnp.float32),
            pltpu.SemaphoreType.DMA,
            pltpu.SemaphoreType.DMA,
            pltpu.SemaphoreType.DMA,
            pltpu.SemaphoreType.DMA,
        ],
    )
    return kern(p, send2d, recv2d, sh2d)


def kernel(node_features, relative_positions_sh, senders, receivers,
           W0, W1, Ws0, Ws1):
    senders = senders.astype(jnp.int32).reshape(N_EDGES // B, B)
    receivers = receivers.astype(jnp.int32).reshape(N_EDGES // B, B)
    sh2d = relative_positions_sh.astype(jnp.float32).T
    wbig = _build_wbig(W0, W1, Ws0, Ws1)
    p = _precompute_p(node_features, wbig)
    parts = _sc_aggregate(p, senders, receivers, sh2d)
    return _epilogue(parts)


# speedup vs baseline: 1.3056x; 1.1034x over previous
import functools

import jax
import jax.numpy as jnp
from jax import lax
from jax.experimental import pallas as pl
from jax.experimental.pallas import tpu as pltpu
from jax.experimental.pallas import tpu_sc as plsc

N_NODES = 10000
N_EDGES = 320000
NC = 2
NS = 16
L = 16
EDGES_PER_WORKER = N_EDGES // (NC * NS)
B = 40
NBLK = EDGES_PER_WORKER // B
SBK = 10
NSB = NBLK // SBK
NZCHUNK = N_NODES // B
ACC_W = 144


def _build_wbig(W0, W1, Ws0, Ws1):
    c1 = 1.0 / (1.5 * jnp.sqrt(96.0))
    c2 = 1.0 / jnp.sqrt(32.0)
    A0, A1, A2 = W0[0:32], W0[32:64], W0[64:96]
    B0, B1, B2 = W1[0:32], W1[32:64], W1[64:96]
    z = jnp.zeros((32, 32), jnp.float32)
    CB = c1 * B0 + c2 * Ws1
    E1 = c1 * B1
    D2 = (c1 / jnp.sqrt(3.0)) * A2
    r_x0 = jnp.concatenate(
        [c1 * A0 + c2 * Ws0, z, z, z, c1 * A1, z, z, z, z, z, z, c1 * B2],
        axis=1)
    rows = [r_x0]
    for j in range(3):
        blocks = [z] * 12
        blocks[1 + j] = CB
        blocks[5 + j] = E1
        blocks[8 + j] = D2
        rows.append(jnp.concatenate(blocks, axis=1))
    return jnp.concatenate(rows, axis=0)


def _matmul_body(x_ref, w_ref, o_ref):
    o_ref[...] = jnp.dot(x_ref[...], w_ref[...],
                         preferred_element_type=jnp.float32)


def _precompute_p(x, wbig):
    blk = 400
    grid = (N_NODES // blk,)
    return pl.pallas_call(
        _matmul_body,
        grid=grid,
        in_specs=[
            pl.BlockSpec((blk, 128), lambda i: (i, 0)),
            pl.BlockSpec((128, 384), lambda i: (0, 0)),
        ],
        out_specs=pl.BlockSpec((blk, 384), lambda i: (i, 0)),
        out_shape=jax.ShapeDtypeStruct((N_NODES, 384), jnp.float32),
    )(x, wbig)


def _epilogue_body(p_ref, o_ref):
    s = p_ref[0] + p_ref[1]
    cnt = jnp.maximum(s[:, 128:129], 1.0)
    o_ref[...] = s[:, 0:128] / cnt


def _epilogue(parts):
    blk = 400
    grid = (N_NODES // blk,)
    return pl.pallas_call(
        _epilogue_body,
        grid=grid,
        in_specs=[pl.BlockSpec((2, blk, ACC_W), lambda i: (0, i, 0))],
        out_specs=pl.BlockSpec((blk, 128), lambda i: (i, 0)),
        out_shape=jax.ShapeDtypeStruct((N_NODES, 128), jnp.float32),
    )(parts)


def _splat(shc, k, col):
    ki = jnp.full((L,), k, jnp.int32)
    ci = jnp.full((L,), col, jnp.int32)
    return plsc.load_gather(shc, [ki, ci])


def _sc_body(p_hbm, send_hbm, recv_hbm, sh_hbm, out_hbm,
             sidxb0, sidxb1, ridxb0, ridxb1, shc, pbuf0, pbuf1, ybuf, acc,
             gsem0, gsem1, isem0, isem1):
    c = lax.axis_index("c")
    s = lax.axis_index("s")
    wid = c * NS + s
    blkbase = wid * NBLK

    zvec = jnp.zeros((L,), jnp.float32)
    for r in range(B):
        for k in range(ACC_W // L):
            ybuf[r, pl.ds(k * L, L)] = zvec

    def zloop(k, carry):
        cidx = k * NS + s

        @pl.when(cidx < NZCHUNK)
        def _():
            pltpu.sync_copy(ybuf, acc.at[pl.ds(cidx * B, B)])
        return carry
    lax.fori_loop(0, (NZCHUNK + NS - 1) // NS, zloop, 0)

    lane = lax.iota(jnp.int32, L)
    ones_chunk = jnp.where(lane == 0, 1.0, 0.0).astype(jnp.float32)
    for r in range(B):
        ybuf[r, pl.ds(128, L)] = ones_chunk
    plsc.subcore_barrier()

    pbufs = (pbuf0, pbuf1)
    gsems = (gsem0, gsem1)
    sidxbs = (sidxb0, sidxb1)
    ridxbs = (ridxb0, ridxb1)
    isems = (isem0, isem1)

    def compute_block(pbuf, j):
        g = [None] * 4
        for e in range(B):
            if e % 4 == 0:
                for k in range(4):
                    g[k] = shc[k, pl.ds(j * B + e, L)]
            lidx = jnp.full((L,), e % 4, jnp.int32)
            sh_l0 = g[0].at[lidx].get(mode="promise_in_bounds")
            s1_a = g[1].at[lidx].get(mode="promise_in_bounds")
            s1_b = g[2].at[lidx].get(mode="promise_in_bounds")
            s1_c = g[3].at[lidx].get(mode="promise_in_bounds")
            t0 = pbuf[e, pl.ds(352, L)]
            t1 = pbuf[e, pl.ds(368, L)]
            for k in range(2):
                off = k * L
                y = (pbuf[e, pl.ds(off, L)]
                     + sh_l0 * pbuf[e, pl.ds(128 + off, L)]
                     + s1_a * pbuf[e, pl.ds(256 + off, L)]
                     + s1_b * pbuf[e, pl.ds(288 + off, L)]
                     + s1_c * pbuf[e, pl.ds(320 + off, L)])
                ybuf[e, pl.ds(off, L)] = y
            for jj, sj in ((0, s1_a), (1, s1_b), (2, s1_c)):
                for m, tm in ((0, t0), (1, t1)):
                    off = 32 + 32 * jj + m * L
                    y = (pbuf[e, pl.ds(off, L)]
                         + sh_l0 * pbuf[e, pl.ds(128 + off, L)]
                         + sj * tm)
                    ybuf[e, pl.ds(off, L)] = y

    def fetch_idx(sb, slot):
        rowbase = blkbase + sb * SBK
        pltpu.async_copy(send_hbm.at[pl.ds(rowbase, SBK)],
                         sidxbs[slot], isems[slot])
        pltpu.async_copy(recv_hbm.at[pl.ds(rowbase, SBK)],
                         ridxbs[slot], isems[slot])

    def wait_idx(sb, slot):
        rowbase = blkbase + sb * SBK
        pltpu.make_async_copy(send_hbm.at[pl.ds(rowbase, SBK)],
                              sidxbs[slot], isems[slot]).wait()
        pltpu.make_async_copy(recv_hbm.at[pl.ds(rowbase, SBK)],
                              ridxbs[slot], isems[slot]).wait()

    fetch_idx(0, 0)

    def sb_exec(sb, slot):
        sidxb = sidxbs[slot]
        ridxb = ridxbs[slot]
        wait_idx(sb, slot)

        @pl.when(sb + 1 < NSB)
        def _():
            fetch_idx(sb + 1, 1 - slot)
        ebase = (blkbase + sb * SBK) * B
        pltpu.sync_copy(sh_hbm.at[:, pl.ds(ebase, SBK * B)],
                        shc.at[:, pl.ds(0, SBK * B)])
        pltpu.async_copy(p_hbm.at[sidxb.at[0]], pbuf0, gsem0)
        pltpu.async_copy(p_hbm.at[sidxb.at[1]], pbuf1, gsem1)

        def jj_body(jj, carry2):
            for phase in range(2):
                j = jj * 2 + phase
                pltpu.make_async_copy(p_hbm.at[sidxb.at[j]],
                                      pbufs[phase], gsems[phase]).wait()
                compute_block(pbufs[phase], j)
                pltpu.sync_copy(ybuf, acc.at[ridxb.at[j]], add=True)

                @pl.when(j + 2 < SBK)
                def _():
                    pltpu.async_copy(p_hbm.at[sidxb.at[j + 2]],
                                     pbufs[phase], gsems[phase])
            return carry2
        lax.fori_loop(0, SBK // 2, jj_body, 0)

    def sb2_body(sb2, carry):
        for slot in range(2):
            sb_exec(sb2 * 2 + slot, slot)
        return carry
    lax.fori_loop(0, NSB // 2, sb2_body, 0)
    sb_exec(jnp.int32(NSB - 1), 0)

    plsc.subcore_barrier()
    rpt = N_NODES // NS
    pltpu.sync_copy(acc.at[pl.ds(s * rpt, rpt)],
                    out_hbm.at[c, pl.ds(s * rpt, rpt)])


def _sc_aggregate(p, send2d, recv2d, sh2d):
    mesh = plsc.VectorSubcoreMesh(core_axis_name="c", subcore_axis_name="s")
    kern = pl.kernel(
        _sc_body,
        out_type=jax.ShapeDtypeStruct((NC, N_NODES, ACC_W), jnp.float32),
        mesh=mesh,
        compiler_params=pltpu.CompilerParams(use_tc_tiling_on_sc=False,
                                             needs_layout_passes=False),
        scratch_types=[
            pltpu.VMEM((SBK, B), jnp.int32),
            pltpu.VMEM((SBK, B), jnp.int32),
            pltpu.VMEM((SBK, B), jnp.int32),
            pltpu.VMEM((SBK, B), jnp.int32),
            pltpu.VMEM((4, SBK * B + L), jnp.float32),
            pltpu.VMEM((B, 384), jnp.float32),
            pltpu.VMEM((B, 384), jnp.float32),
            pltpu.VMEM((B, ACC_W), jnp.float32),
            pltpu.VMEM_SHARED((N_NODES, ACC_W), jnp.float32),
            pltpu.SemaphoreType.DMA,
            pltpu.SemaphoreType.DMA,
            pltpu.SemaphoreType.DMA,
            pltpu.SemaphoreType.DMA,
        ],
    )
    return kern(p, send2d, recv2d, sh2d)


def kernel(node_features, relative_positions_sh, senders, receivers,
           W0, W1, Ws0, Ws1):
    senders = senders.astype(jnp.int32).reshape(N_EDGES // B, B)
    receivers = receivers.astype(jnp.int32).reshape(N_EDGES // B, B)
    sh2d = relative_positions_sh.astype(jnp.float32).T
    wbig = _build_wbig(W0, W1, Ws0, Ws1)
    p = _precompute_p(node_features, wbig)
    parts = _sc_aggregate(p, senders, receivers, sh2d)
    return _epilogue(parts)
